# baseline (speedup 1.0000x reference)
import jax
import jax.numpy as jnp
from jax.experimental import pallas as pl
from jax.experimental.pallas import tpu as pltpu

_BN = 400


def _body(nei_ref, h_ref, wm_ref, bm_ref, w1_ref, w2_ref, b_ref, out_ref):
    bn, deg = h_ref.shape[0], nei_ref.shape[0] // h_ref.shape[0]
    d = nei_ref.shape[1]
    x = jnp.dot(nei_ref[...].astype(jnp.bfloat16), wm_ref[...].astype(jnp.bfloat16),
                preferred_element_type=jnp.float32)
    m = jnp.maximum(jnp.max(x.reshape(bn, deg, d), axis=1) + bm_ref[...], 0.0)
    out = jnp.dot(m, w1_ref[...], preferred_element_type=jnp.float32)
    out = out + jnp.dot(h_ref[...], w2_ref[...], preferred_element_type=jnp.float32)
    out_ref[...] = out + b_ref[...]


def kernel(h, nei, W_mlp, b_mlp, W, b):
    n, d = h.shape
    deg = nei.shape[1]
    out_dim = W.shape[0]
    bn = _BN
    nei2 = nei.reshape(n * deg, d)
    wm = W_mlp.T
    w1 = W[:, :d].T
    w2 = W[:, d:].T
    bm = b_mlp.reshape(1, d)
    bb = b.reshape(1, out_dim)
    return pl.pallas_call(
        _body,
        grid=(n // bn,),
        in_specs=[
            pl.BlockSpec((bn * deg, d), lambda i: (i, 0)),
            pl.BlockSpec((bn, d), lambda i: (i, 0)),
            pl.BlockSpec((d, d), lambda i: (0, 0)),
            pl.BlockSpec((1, d), lambda i: (0, 0)),
            pl.BlockSpec((d, out_dim), lambda i: (0, 0)),
            pl.BlockSpec((d, out_dim), lambda i: (0, 0)),
            pl.BlockSpec((1, out_dim), lambda i: (0, 0)),
        ],
        out_specs=pl.BlockSpec((bn, out_dim), lambda i: (i, 0)),
        out_shape=jax.ShapeDtypeStruct((n, out_dim), jnp.float32),
        compiler_params=pltpu.CompilerParams(
            dimension_semantics=("parallel",),
        ),
    )(nei2, h, wm, bm, w1, w2, bb)


# speedup vs baseline: 1.0523x; 1.0523x over previous
import jax
import jax.numpy as jnp
from jax.experimental import pallas as pl
from jax.experimental.pallas import tpu as pltpu

_CH = 1000
_NB = 2


def _body(nei_hbm, h_hbm, wm_v, bm_v, w1_v, w2_v, b_v, out_hbm, *scr):
    nb = _NB
    nei_bufs = scr[0:nb]
    h_bufs = scr[nb:2 * nb]
    out_bufs = scr[2 * nb:3 * nb]
    nei_sem, h_sem, out_sem = scr[3 * nb:3 * nb + 3]
    ch, deg = h_bufs[0].shape[0], nei_bufs[0].shape[0] // h_bufs[0].shape[0]
    d = nei_bufs[0].shape[1]
    chd = ch * deg
    steps = nei_hbm.shape[0] // chd
    outer = steps // nb

    def in_copies(c, b):
        return (
            pltpu.make_async_copy(nei_hbm.at[pl.ds(c * chd, chd), :],
                                  nei_bufs[b], nei_sem.at[b]),
            pltpu.make_async_copy(h_hbm.at[pl.ds(c * ch, ch), :],
                                  h_bufs[b], h_sem.at[b]),
        )

    for b in range(nb):
        for cp in in_copies(b, b):
            cp.start()

    wm = wm_v[...].astype(jnp.bfloat16)

    def outer_body(g, carry):
        for b in range(nb):
            c = g * nb + b
            for cp in in_copies(c, b):
                cp.wait()
            x = jnp.dot(nei_bufs[b][...].astype(jnp.bfloat16), wm,
                        preferred_element_type=jnp.float32)
            m = jnp.max(x.reshape(ch, deg, d), axis=1)
            m = jnp.maximum(m + bm_v[...], 0.0)
            o = jnp.dot(m, w1_v[...], preferred_element_type=jnp.float32)
            o = o + jnp.dot(h_bufs[b][...], w2_v[...],
                            preferred_element_type=jnp.float32)

            @pl.when(g > 0)
            def _():
                pltpu.make_async_copy(
                    out_bufs[b], out_hbm.at[pl.ds((c - nb) * ch, ch), :],
                    out_sem.at[b]).wait()

            out_bufs[b][...] = o + b_v[...]
            pltpu.make_async_copy(out_bufs[b],
                                  out_hbm.at[pl.ds(c * ch, ch), :],
                                  out_sem.at[b]).start()

            @pl.when(g < outer - 1)
            def _():
                for cp in in_copies(c + nb, b):
                    cp.start()
        return carry

    jax.lax.fori_loop(0, outer, outer_body, 0)

    for b in range(nb):
        c = (outer - 1) * nb + b
        pltpu.make_async_copy(out_bufs[b], out_hbm.at[pl.ds(c * ch, ch), :],
                              out_sem.at[b]).wait()


def kernel(h, nei, W_mlp, b_mlp, W, b):
    n, d = h.shape
    deg = nei.shape[1]
    out_dim = W.shape[0]
    ch, nb = _CH, _NB
    nei2 = nei.reshape(n * deg, d)
    wm = W_mlp.T
    w1 = W[:, :d].T
    w2 = W[:, d:].T
    bm = b_mlp.reshape(1, d)
    bb = b.reshape(1, out_dim)
    hbm = pl.BlockSpec(memory_space=pltpu.MemorySpace.HBM)
    vmem = pl.BlockSpec(memory_space=pltpu.MemorySpace.VMEM)
    scratch = (
        [pltpu.VMEM((ch * deg, d), jnp.float32) for _ in range(nb)]
        + [pltpu.VMEM((ch, d), jnp.float32) for _ in range(nb)]
        + [pltpu.VMEM((ch, out_dim), jnp.float32) for _ in range(nb)]
        + [pltpu.SemaphoreType.DMA((nb,))] * 3
    )
    return pl.pallas_call(
        _body,
        in_specs=[hbm, hbm, vmem, vmem, vmem, vmem, vmem],
        out_specs=hbm,
        out_shape=jax.ShapeDtypeStruct((n, out_dim), jnp.float32),
        scratch_shapes=scratch,
    )(nei2, h, wm, bm, w1, w2, bb)


# speedup vs baseline: 1.1101x; 1.0550x over previous
import jax
import jax.numpy as jnp
from jax.experimental import pallas as pl
from jax.experimental.pallas import tpu as pltpu

_BN = 1000


def _body(nei_ref, h_ref, wm_ref, bm_ref, w1_ref, w2_ref, b_ref, out_ref):
    bn, deg = h_ref.shape[0], nei_ref.shape[0] // h_ref.shape[0]
    d = nei_ref.shape[1]
    x = jnp.dot(nei_ref[...].astype(jnp.bfloat16), wm_ref[...].astype(jnp.bfloat16),
                preferred_element_type=jnp.float32)
    m = jnp.maximum(jnp.max(x.reshape(bn, deg, d), axis=1) + bm_ref[...], 0.0)
    out = jnp.dot(m, w1_ref[...], preferred_element_type=jnp.float32)
    out = out + jnp.dot(h_ref[...], w2_ref[...], preferred_element_type=jnp.float32)
    out_ref[...] = out + b_ref[...]


def kernel(h, nei, W_mlp, b_mlp, W, b):
    n, d = h.shape
    deg = nei.shape[1]
    out_dim = W.shape[0]
    bn = _BN
    nei2 = nei.reshape(n * deg, d)
    wm = W_mlp.T
    w1 = W[:, :d].T
    w2 = W[:, d:].T
    bm = b_mlp.reshape(1, d)
    bb = b.reshape(1, out_dim)
    return pl.pallas_call(
        _body,
        grid=(n // bn,),
        in_specs=[
            pl.BlockSpec((bn * deg, d), lambda i: (i, 0)),
            pl.BlockSpec((bn, d), lambda i: (i, 0)),
            pl.BlockSpec((d, d), lambda i: (0, 0)),
            pl.BlockSpec((1, d), lambda i: (0, 0)),
            pl.BlockSpec((d, out_dim), lambda i: (0, 0)),
            pl.BlockSpec((d, out_dim), lambda i: (0, 0)),
            pl.BlockSpec((1, out_dim), lambda i: (0, 0)),
        ],
        out_specs=pl.BlockSpec((bn, out_dim), lambda i: (i, 0)),
        out_shape=jax.ShapeDtypeStruct((n, out_dim), jnp.float32),
        compiler_params=pltpu.CompilerParams(
            dimension_semantics=("parallel",),
        ),
    )(nei2, h, wm, bm, w1, w2, bb)
